# Initial kernel scaffold; baseline (speedup 1.0000x reference)
import jax
import jax.numpy as jnp
from jax.experimental import pallas as pl


def kernel(state, action, w0_s, w0_a, hidden_w0, w_last, biases, b_last):
    raise NotImplementedError("write your pallas kernel here")


# speedup vs baseline: 1.0602x; 1.0602x over previous
import functools

import jax
import jax.numpy as jnp
from jax import lax
from jax.experimental import pallas as pl
from jax.experimental.pallas import tpu as pltpu

_TILE_B = 1024


def _critic_kernel(state_ref, action_ref, w0s_ref, w0a_ref, w1a_ref, w1b_ref,
                   wl1_ref, wl2_ref, b_ref, b_last_ref, out_ref, *, H):
    b = b_ref[...]

    h = (jnp.dot(state_ref[...], w0s_ref[...], preferred_element_type=jnp.float32)
         + jnp.dot(action_ref[...], w0a_ref[...], preferred_element_type=jnp.float32)
         + b[0:1, :])
    h = jnp.maximum(h, 0.0)

    g1 = jnp.maximum(
        jnp.dot(h[:, :H], w1a_ref[...], preferred_element_type=jnp.float32)
        + b[1:2, :H], 0.0)
    g2 = jnp.maximum(
        jnp.dot(h[:, H:], w1b_ref[...], preferred_element_type=jnp.float32)
        + b[1:2, H:], 0.0)

    q1 = lax.dot_general(wl1_ref[...], g1,
                         dimension_numbers=(((0,), (1,)), ((), ())),
                         preferred_element_type=jnp.float32)
    q2 = lax.dot_general(wl2_ref[...], g2,
                         dimension_numbers=(((0,), (1,)), ((), ())),
                         preferred_element_type=jnp.float32)
    out_ref[...] = jnp.concatenate([q1, q2], axis=0) + b_last_ref[...]


@jax.jit
def _forward(state, action, w0_s, w0_a, hidden_w0, w_last, biases, b_last):
    B, S = state.shape
    A = action.shape[1]
    H = hidden_w0.shape[0] // 2

    w1a = hidden_w0[:H, :H]
    w1b = hidden_w0[H:, H:]
    wl1 = w_last[:H, 0:1]
    wl2 = w_last[H:, 1:2]

    tile = _TILE_B
    num_tiles = pl.cdiv(B, tile)
    b_pad = num_tiles * tile
    if b_pad != B:
        state = jnp.pad(state, ((0, b_pad - B), (0, 0)))
        action = jnp.pad(action, ((0, b_pad - B), (0, 0)))

    weight_inputs = [w0_s, w0_a, w1a, w1b, wl1, wl2, biases, b_last]

    def resident(arr):
        nd = arr.ndim
        return pl.BlockSpec(arr.shape, lambda i: (0,) * nd)

    qs = pl.pallas_call(
        functools.partial(_critic_kernel, H=H),
        grid=(num_tiles,),
        out_shape=jax.ShapeDtypeStruct((2, b_pad), jnp.float32),
        in_specs=[
            pl.BlockSpec((tile, S), lambda i: (i, 0)),
            pl.BlockSpec((tile, A), lambda i: (i, 0)),
            *[resident(w) for w in weight_inputs],
        ],
        out_specs=pl.BlockSpec((2, tile), lambda i: (0, i)),
        compiler_params=pltpu.CompilerParams(
            dimension_semantics=("parallel",)),
    )(state, action, *weight_inputs)

    return qs[0, :B].reshape(B, 1), qs[1, :B].reshape(B, 1)


def kernel(state, action, w0_s, w0_a, hidden_w0, w_last, biases, b_last):
    return _forward(state, action, w0_s, w0_a, hidden_w0, w_last, biases,
                    b_last)
